# Initial kernel scaffold; baseline (speedup 1.0000x reference)
import jax
import jax.numpy as jnp
from jax.experimental import pallas as pl


def kernel(x, edge_index, W1, b1, W2, b2):
    raise NotImplementedError("write your pallas kernel here")


# speedup vs baseline: 7.4079x; 7.4079x over previous
import functools

import jax
import jax.numpy as jnp
from jax import lax
from jax.experimental import pallas as pl
from jax.experimental.pallas import tpu as pltpu
from jax.experimental.pallas import tpu_sc as plsc

N = 10000
E = 160000
NS = 16
NC = 2
EDGES_PER_SUB = E // NS
RC = 80
EDGES_PER_TILE = E // (NC * NS)
EC = 80
EC2 = 40
BR = 1000
GI = N // BR

_mesh = plsc.VectorSubcoreMesh(core_axis_name="c", subcore_axis_name="s")


def _sc_degree(dst, ones):
    @functools.partial(
        pl.kernel,
        out_type=jax.ShapeDtypeStruct((2 * N, 16), jnp.float32),
        mesh=_mesh,
        scratch_types=[
            pltpu.VMEM((EC2,), jnp.int32),
            pltpu.VMEM((EC2, 16), jnp.float32),
            pltpu.VMEM_SHARED((N, 16), jnp.float32),
        ],
    )
    def k(dst_hbm, ones_hbm, out_hbm, didx, ones_v, acc_sh):
        cid = lax.axis_index("c")
        sid = lax.axis_index("s")

        @pl.loop(sid * RC, N, step=NS * RC)
        def _(r0):
            pltpu.sync_copy(ones_hbm.at[pl.ds(r0, RC)],
                            acc_sh.at[pl.ds(r0, RC)])

        pltpu.sync_copy(ones_hbm.at[pl.ds(0, EC2)], ones_v)
        plsc.subcore_barrier()

        base = (cid * NS + sid) * EDGES_PER_TILE

        @pl.loop(0, EDGES_PER_TILE, step=EC2)
        def _(j):
            pltpu.sync_copy(dst_hbm.at[pl.ds(base + j, EC2)], didx)
            pltpu.sync_copy(ones_v, acc_sh.at[didx], add=True)

        plsc.subcore_barrier()

        @pl.loop(sid * RC, N, step=NS * RC)
        def _(r0):
            pltpu.sync_copy(acc_sh.at[pl.ds(r0, RC)],
                            out_hbm.at[pl.ds(cid * N + r0, RC)])

    return k(dst, ones)


def _sc_aggregate(g, src2, dst):
    @functools.partial(
        pl.kernel,
        out_type=jax.ShapeDtypeStruct((2 * N, 128), jnp.float32),
        mesh=_mesh,
        scratch_types=[
            pltpu.VMEM((EC,), jnp.int32),
            pltpu.VMEM((EC,), jnp.int32),
            pltpu.VMEM((EC, 128), jnp.float32),
            pltpu.VMEM_SHARED((N, 128), jnp.float32),
        ],
    )
    def k(g_hbm, src_hbm, dst_hbm, out_hbm, sidx, didx, rows, acc_sh):
        cid = lax.axis_index("c")
        sid = lax.axis_index("s")

        @pl.loop(sid * RC, N, step=NS * RC)
        def _(r0):
            pltpu.sync_copy(g_hbm.at[pl.ds(cid * N + r0, RC)],
                            acc_sh.at[pl.ds(r0, RC)])

        plsc.subcore_barrier()

        sbase = cid * E + sid * EDGES_PER_SUB
        dbase = sid * EDGES_PER_SUB

        @pl.loop(0, EDGES_PER_SUB, step=EC)
        def _(j):
            pltpu.sync_copy(src_hbm.at[pl.ds(sbase + j, EC)], sidx)
            pltpu.sync_copy(dst_hbm.at[pl.ds(dbase + j, EC)], didx)
            pltpu.sync_copy(g_hbm.at[sidx], rows)
            pltpu.sync_copy(rows, acc_sh.at[didx], add=True)

        plsc.subcore_barrier()

        @pl.loop(sid * RC, N, step=NS * RC)
        def _(r0):
            pltpu.sync_copy(acc_sh.at[pl.ds(r0, RC)],
                            out_hbm.at[pl.ds(cid * N + r0, RC)])

    return k(g, src2, dst)


def _dinv_block(p_lo, p_hi):
    return lax.rsqrt(p_lo[:, 0:1] + p_hi[:, 0:1] - 1.0)


def _tc1_body(x_ref, w1_ref, plo_ref, phi_ref, g_ref):
    dinv = _dinv_block(plo_ref[...], phi_ref[...])
    h = lax.dot_general(x_ref[...], w1_ref[...], (((1,), (0,)), ((), ())),
                        preferred_element_type=jnp.float32)
    g_ref[...] = h * dinv


def _tc1(x, W1, degp):
    return pl.pallas_call(
        _tc1_body,
        grid=(GI, 2),
        in_specs=[
            pl.BlockSpec((BR, 512), lambda i, c: (i, 0)),
            pl.BlockSpec((512, 128), lambda i, c: (0, c)),
            pl.BlockSpec((BR, 16), lambda i, c: (i, 0)),
            pl.BlockSpec((BR, 16), lambda i, c: (i + GI, 0)),
        ],
        out_specs=pl.BlockSpec((BR, 128), lambda i, c: (c * GI + i, 0)),
        out_shape=jax.ShapeDtypeStruct((2 * N, 128), jnp.float32),
    )(x, W1, degp, degp)


def _tc2_body(alo_ref, ahi_ref, glo_ref, ghi_ref, plo_ref, phi_ref,
              w2a_ref, w2b_ref, blo_ref, bhi_ref, g2_ref):
    dinv = _dinv_block(plo_ref[...], phi_ref[...])
    a0 = jnp.maximum((alo_ref[...] + glo_ref[...]) * dinv + blo_ref[0], 0.0)
    a1 = jnp.maximum((ahi_ref[...] + ghi_ref[...]) * dinv + bhi_ref[0], 0.0)
    h2 = (lax.dot_general(a0, w2a_ref[...], (((1,), (0,)), ((), ())),
                          preferred_element_type=jnp.float32)
          + lax.dot_general(a1, w2b_ref[...], (((1,), (0,)), ((), ())),
                            preferred_element_type=jnp.float32))
    g2_ref[...] = h2 * dinv


def _tc2(acc1, g1, degp, b1, W2):
    b1r = b1.reshape(2, 1, 128)
    return pl.pallas_call(
        _tc2_body,
        grid=(GI, 2),
        in_specs=[
            pl.BlockSpec((BR, 128), lambda i, c: (i, 0)),
            pl.BlockSpec((BR, 128), lambda i, c: (i + GI, 0)),
            pl.BlockSpec((BR, 128), lambda i, c: (i, 0)),
            pl.BlockSpec((BR, 128), lambda i, c: (i + GI, 0)),
            pl.BlockSpec((BR, 16), lambda i, c: (i, 0)),
            pl.BlockSpec((BR, 16), lambda i, c: (i + GI, 0)),
            pl.BlockSpec((128, 128), lambda i, c: (0, c)),
            pl.BlockSpec((128, 128), lambda i, c: (1, c)),
            pl.BlockSpec((1, 1, 128), lambda i, c: (0, 0, 0)),
            pl.BlockSpec((1, 1, 128), lambda i, c: (1, 0, 0)),
        ],
        out_specs=pl.BlockSpec((BR, 128), lambda i, c: (c * GI + i, 0)),
        out_shape=jax.ShapeDtypeStruct((2 * N, 128), jnp.float32),
    )(acc1, acc1, g1, g1, degp, degp, W2, W2, b1r, b1r)


def _tc3_body(acc_ref, g_ref, plo_ref, phi_ref, b_ref, out_ref):
    dinv = _dinv_block(plo_ref[...], phi_ref[...])
    out_ref[...] = jnp.maximum((acc_ref[...] + g_ref[...]) * dinv + b_ref[0],
                               0.0)


def _tc3(acc2, g2, degp, b2):
    b2r = b2.reshape(2, 1, 128)
    return pl.pallas_call(
        _tc3_body,
        grid=(GI, 2),
        in_specs=[
            pl.BlockSpec((BR, 128), lambda i, c: (c * GI + i, 0)),
            pl.BlockSpec((BR, 128), lambda i, c: (c * GI + i, 0)),
            pl.BlockSpec((BR, 16), lambda i, c: (i, 0)),
            pl.BlockSpec((BR, 16), lambda i, c: (i + GI, 0)),
            pl.BlockSpec((1, 1, 128), lambda i, c: (c, 0, 0)),
        ],
        out_specs=pl.BlockSpec((BR, 128), lambda i, c: (i, c)),
        out_shape=jax.ShapeDtypeStruct((N, 256), jnp.float32),
    )(acc2, g2, degp, degp, b2r)


def kernel(x, edge_index, W1, b1, W2, b2):
    src = edge_index[0]
    dst = edge_index[1]
    src2 = jnp.concatenate([src, src + N])
    ones = jnp.ones((N, 16), jnp.float32)

    degp = _sc_degree(dst, ones)
    g1 = _tc1(x, W1, degp)
    acc1 = _sc_aggregate(g1, src2, dst)
    g2 = _tc2(acc1, g1, degp, b1, W2)
    acc2 = _sc_aggregate(g2, src2, dst)
    return _tc3(acc2, g2, degp, b2)


# speedup vs baseline: 15.3864x; 2.0770x over previous
import functools

import jax
import jax.numpy as jnp
from jax import lax
from jax.experimental import pallas as pl
from jax.experimental.pallas import tpu as pltpu
from jax.experimental.pallas import tpu_sc as plsc

N = 10000
E = 160000
NS = 16
NC = 2
EDGES_PER_SUB = E // NS
RC = 80
EDGES_PER_TILE = E // (NC * NS)
EC = 100
DRA = 5
RCH = EDGES_PER_SUB // (EC * DRA)
EC2 = 40
NCH2 = EDGES_PER_TILE // EC2
BR = 1000
GI = N // BR

_mesh = plsc.VectorSubcoreMesh(core_axis_name="c", subcore_axis_name="s")


def _sc_degree(dst, ones):
    DR = 5
    RCH2 = NCH2 // DR
    dstr = dst.reshape(NC * NS * DR, RCH2, EC2)

    @functools.partial(
        pl.kernel,
        out_type=jax.ShapeDtypeStruct((2 * N, 16), jnp.float32),
        mesh=_mesh,
        scratch_types=[
            pltpu.VMEM((RCH2, EC2), jnp.int32),
            pltpu.VMEM((EC2, 16), jnp.float32),
            pltpu.VMEM_SHARED((N, 16), jnp.float32),
        ],
    )
    def k(dst_hbm, ones_hbm, out_hbm, didx, ones_v, acc_sh):
        cid = lax.axis_index("c")
        sid = lax.axis_index("s")

        pltpu.sync_copy(ones_hbm.at[pl.ds(0, EC2)], ones_v)

        @pl.loop(sid * RC, N, step=NS * RC)
        def _(r0):
            pltpu.sync_copy(ones_hbm.at[pl.ds(r0, RC)],
                            acc_sh.at[pl.ds(r0, RC)])

        plsc.subcore_barrier()

        @pl.loop(0, DR)
        def _(r):
            pltpu.sync_copy(dst_hbm.at[(cid * NS + sid) * DR + r], didx)

            @pl.loop(0, RCH2)
            def _(j):
                pltpu.sync_copy(ones_v, acc_sh.at[didx.at[j]], add=True)

        plsc.subcore_barrier()

        @pl.loop(sid * RC, N, step=NS * RC)
        def _(r0):
            pltpu.sync_copy(acc_sh.at[pl.ds(r0, RC)],
                            out_hbm.at[pl.ds(cid * N + r0, RC)])

    return k(dstr, ones)


def _sc_aggregate(g, src2, dst):
    srcr = src2.reshape(NC * NS * DRA, RCH, EC)
    dstr = dst.reshape(NS * DRA, RCH, EC)

    @functools.partial(
        pl.kernel,
        out_type=jax.ShapeDtypeStruct((2 * N, 128), jnp.float32),
        mesh=_mesh,
        scratch_types=[
            pltpu.VMEM((RCH, EC), jnp.int32),
            pltpu.VMEM((RCH, EC), jnp.int32),
            pltpu.VMEM((EC, 128), jnp.float32),
            pltpu.VMEM((EC, 128), jnp.float32),
            pltpu.VMEM_SHARED((N, 128), jnp.float32),
            pltpu.SemaphoreType.DMA,
            pltpu.SemaphoreType.DMA,
        ],
    )
    def k(g_hbm, src_hbm, dst_hbm, out_hbm, sidx, didx, rows0, rows1,
          acc_sh, sem0, sem1):
        cid = lax.axis_index("c")
        sid = lax.axis_index("s")

        @pl.loop(sid * RC, N, step=NS * RC)
        def _(r0):
            pltpu.sync_copy(g_hbm.at[pl.ds(cid * N + r0, RC)],
                            acc_sh.at[pl.ds(r0, RC)])

        plsc.subcore_barrier()

        def gstart(j, rows, sem):
            pltpu.async_copy(g_hbm.at[sidx.at[j]], rows, sem)

        def gwait(j, rows, sem):
            pltpu.make_async_copy(g_hbm.at[sidx.at[j]], rows, sem).wait()

        def scat(j, rows):
            pltpu.sync_copy(rows, acc_sh.at[didx.at[j]], add=True)

        @pl.loop(0, DRA)
        def _(r):
            pltpu.sync_copy(src_hbm.at[(cid * NS + sid) * DRA + r], sidx)
            pltpu.sync_copy(dst_hbm.at[sid * DRA + r], didx)
            gstart(0, rows0, sem0)

            @pl.loop(0, RCH - 2, step=2)
            def _(j):
                gstart(j + 1, rows1, sem1)
                gwait(j, rows0, sem0)
                scat(j, rows0)
                gstart(j + 2, rows0, sem0)
                gwait(j + 1, rows1, sem1)
                scat(j + 1, rows1)

            gstart(RCH - 1, rows1, sem1)
            gwait(RCH - 2, rows0, sem0)
            scat(RCH - 2, rows0)
            gwait(RCH - 1, rows1, sem1)
            scat(RCH - 1, rows1)

        plsc.subcore_barrier()

        @pl.loop(sid * RC, N, step=NS * RC)
        def _(r0):
            pltpu.sync_copy(acc_sh.at[pl.ds(r0, RC)],
                            out_hbm.at[pl.ds(cid * N + r0, RC)])

    return k(g, srcr, dstr)


def _dinv_block(p_lo, p_hi):
    return lax.rsqrt(p_lo[:, 0:1] + p_hi[:, 0:1] - 1.0)


def _tc1_body(x_ref, w1_ref, plo_ref, phi_ref, g_ref):
    dinv = _dinv_block(plo_ref[...], phi_ref[...])
    h = lax.dot_general(x_ref[...], w1_ref[...], (((1,), (0,)), ((), ())),
                        preferred_element_type=jnp.float32)
    g_ref[...] = h * dinv


def _tc1(x, W1, degp):
    return pl.pallas_call(
        _tc1_body,
        grid=(GI, 2),
        in_specs=[
            pl.BlockSpec((BR, 512), lambda i, c: (i, 0)),
            pl.BlockSpec((512, 128), lambda i, c: (0, c)),
            pl.BlockSpec((BR, 16), lambda i, c: (i, 0)),
            pl.BlockSpec((BR, 16), lambda i, c: (i + GI, 0)),
        ],
        out_specs=pl.BlockSpec((BR, 128), lambda i, c: (c * GI + i, 0)),
        out_shape=jax.ShapeDtypeStruct((2 * N, 128), jnp.float32),
    )(x, W1, degp, degp)


def _tc2_body(alo_ref, ahi_ref, glo_ref, ghi_ref, plo_ref, phi_ref,
              w2a_ref, w2b_ref, blo_ref, bhi_ref, g2_ref):
    dinv = _dinv_block(plo_ref[...], phi_ref[...])
    a0 = jnp.maximum((alo_ref[...] + glo_ref[...]) * dinv + blo_ref[0], 0.0)
    a1 = jnp.maximum((ahi_ref[...] + ghi_ref[...]) * dinv + bhi_ref[0], 0.0)
    h2 = (lax.dot_general(a0, w2a_ref[...], (((1,), (0,)), ((), ())),
                          preferred_element_type=jnp.float32)
          + lax.dot_general(a1, w2b_ref[...], (((1,), (0,)), ((), ())),
                            preferred_element_type=jnp.float32))
    g2_ref[...] = h2 * dinv


def _tc2(acc1, g1, degp, b1, W2):
    b1r = b1.reshape(2, 1, 128)
    return pl.pallas_call(
        _tc2_body,
        grid=(GI, 2),
        in_specs=[
            pl.BlockSpec((BR, 128), lambda i, c: (i, 0)),
            pl.BlockSpec((BR, 128), lambda i, c: (i + GI, 0)),
            pl.BlockSpec((BR, 128), lambda i, c: (i, 0)),
            pl.BlockSpec((BR, 128), lambda i, c: (i + GI, 0)),
            pl.BlockSpec((BR, 16), lambda i, c: (i, 0)),
            pl.BlockSpec((BR, 16), lambda i, c: (i + GI, 0)),
            pl.BlockSpec((128, 128), lambda i, c: (0, c)),
            pl.BlockSpec((128, 128), lambda i, c: (1, c)),
            pl.BlockSpec((1, 1, 128), lambda i, c: (0, 0, 0)),
            pl.BlockSpec((1, 1, 128), lambda i, c: (1, 0, 0)),
        ],
        out_specs=pl.BlockSpec((BR, 128), lambda i, c: (c * GI + i, 0)),
        out_shape=jax.ShapeDtypeStruct((2 * N, 128), jnp.float32),
    )(acc1, acc1, g1, g1, degp, degp, W2, W2, b1r, b1r)


def _tc3_body(acc_ref, g_ref, plo_ref, phi_ref, b_ref, out_ref):
    dinv = _dinv_block(plo_ref[...], phi_ref[...])
    out_ref[...] = jnp.maximum((acc_ref[...] + g_ref[...]) * dinv + b_ref[0],
                               0.0)


def _tc3(acc2, g2, degp, b2):
    b2r = b2.reshape(2, 1, 128)
    return pl.pallas_call(
        _tc3_body,
        grid=(GI, 2),
        in_specs=[
            pl.BlockSpec((BR, 128), lambda i, c: (c * GI + i, 0)),
            pl.BlockSpec((BR, 128), lambda i, c: (c * GI + i, 0)),
            pl.BlockSpec((BR, 16), lambda i, c: (i, 0)),
            pl.BlockSpec((BR, 16), lambda i, c: (i + GI, 0)),
            pl.BlockSpec((1, 1, 128), lambda i, c: (c, 0, 0)),
        ],
        out_specs=pl.BlockSpec((BR, 128), lambda i, c: (i, c)),
        out_shape=jax.ShapeDtypeStruct((N, 256), jnp.float32),
    )(acc2, g2, degp, degp, b2r)


def kernel(x, edge_index, W1, b1, W2, b2):
    src = edge_index[0]
    dst = edge_index[1]
    src2 = jnp.concatenate([src, src + N])
    ones = jnp.ones((N, 16), jnp.float32)

    degp = _sc_degree(dst, ones)
    g1 = _tc1(x, W1, degp)
    acc1 = _sc_aggregate(g1, src2, dst)
    g2 = _tc2(acc1, g1, degp, b1, W2)
    acc2 = _sc_aggregate(g2, src2, dst)
    return _tc3(acc2, g2, degp, b2)
